# baseline (speedup 1.0000x reference)
import functools

import jax
import jax.numpy as jnp
from jax import lax
from jax.experimental import pallas as pl
from jax.experimental.pallas import tpu as pltpu
from jax.experimental.pallas import tpu_sc as plsc

B = 16384
K = 32
KH = K // 2
N_ROWS = 1000000
NC = 2
NS = 16
NW = NC * NS
BPW = B // NW
C = 16
NCH = BPW // C
L = 16


_mesh = plsc.VectorSubcoreMesh(core_axis_name="c", subcore_axis_name="s")


@functools.partial(
    pl.kernel,
    mesh=_mesh,
    compiler_params=pltpu.CompilerParams(needs_layout_passes=False),
    out_type=jax.ShapeDtypeStruct((B,), jnp.float32),
    scratch_types=[
        pltpu.VMEM((BPW,), jnp.int32),
        pltpu.VMEM((BPW,), jnp.int32),
        pltpu.VMEM((C, KH, 128), jnp.float32),
        pltpu.VMEM((C, KH, 128), jnp.float32),
        pltpu.VMEM((BPW,), jnp.float32),
        pltpu.SemaphoreType.DMA,
    ],
)
def _pmf_sc(user_hbm, item_hbm, wu_t_hbm, wi_t_hbm, out_hbm,
            uvec, ivec, ublk, iblk, oacc, sem):
    wid = lax.axis_index("s") * NC + lax.axis_index("c")
    base = wid * BPW

    pltpu.sync_copy(user_hbm.at[pl.ds(base, BPW)], uvec)
    pltpu.sync_copy(item_hbm.at[pl.ds(base, BPW)], ivec)

    def chunk(c, carry):
        ulane = jnp.bitwise_and(uvec[pl.ds(c * C, L)], 127)
        ilane = jnp.bitwise_and(ivec[pl.ds(c * C, L)], 127)
        ubase = lax.shift_left(
            lax.shift_right_logical(uvec[pl.ds(c * C, L)], 7), 7)
        ibase = lax.shift_left(
            lax.shift_right_logical(ivec[pl.ds(c * C, L)], 7), 7)
        sel = lax.iota(jnp.int32, L)
        zero = jnp.zeros((L,), jnp.int32)
        acc = jnp.zeros((L,), jnp.float32)

        ubs = [pl.multiple_of(jnp.sum(jnp.where(sel == i, ubase, zero)), 128)
               for i in range(C)]
        ibs = [pl.multiple_of(jnp.sum(jnp.where(sel == i, ibase, zero)), 128)
               for i in range(C)]

        for kh in range(K // KH):
            copies = []
            for i in range(C):
                copies.append(pltpu.async_copy(
                    wu_t_hbm.at[pl.ds(kh * KH, KH), pl.ds(ubs[i], 128)],
                    ublk.at[i], sem))
                copies.append(pltpu.async_copy(
                    wi_t_hbm.at[pl.ds(kh * KH, KH), pl.ds(ibs[i], 128)],
                    iblk.at[i], sem))
            for cp in copies:
                cp.wait()

            for kk in range(KH):
                kvec = jnp.full((L,), kk, jnp.int32)
                u = plsc.load_gather(ublk, [sel, kvec, ulane])
                v = plsc.load_gather(iblk, [sel, kvec, ilane])
                acc = acc + u * v

        oacc[pl.ds(c * C, L)] = acc
        return carry

    lax.fori_loop(0, NCH, chunk, 0)

    pltpu.sync_copy(oacc, out_hbm.at[pl.ds(base, BPW)])


def kernel(user, item, W_user, W_item):
    return _pmf_sc(user, item, W_user.T, W_item.T)
